# baseline (speedup 1.0000x reference)
import functools

import jax
import jax.numpy as jnp
from jax import lax
from jax.experimental import pallas as pl
from jax.experimental.pallas import tpu as pltpu
from jax.experimental.pallas import tpu_sc as plsc

BATCH = 16384
VOCAB = 100000
D = 128
OD = D + 1
NC = 2
NS = 16
NW = NC * NS
ROWS_PER_W = BATCH // NW
CHUNK = 128
NCHUNK = ROWS_PER_W // CHUNK
LANES = 16


def _sc_body(idx_hbm, expr_hbm, tab_hbm, tok_hbm, out_hbm,
             idx_v, tok_v,
             rows0, rows1, obuf0, obuf1,
             gsem0, gsem1, ssem0, ssem1, esem):
    rows = (rows0, rows1)
    obuf = (obuf0, obuf1)
    gsem = (gsem0, gsem1)
    ssem = (ssem0, ssem1)

    wid = lax.axis_index("s") * NC + lax.axis_index("c")
    gc0 = wid * NCHUNK
    row0 = wid * ROWS_PER_W

    pltpu.sync_copy(idx_hbm.at[pl.ds(gc0, NCHUNK)], idx_v)
    pltpu.sync_copy(tok_hbm, tok_v)

    ecp = pltpu.async_copy(
        expr_hbm.at[pl.ds(row0, ROWS_PER_W)],
        out_hbm.at[pl.ds(row0, ROWS_PER_W), pl.ds(D, 1)],
        esem)

    tok = [tok_v[pl.ds(j * LANES, LANES)] for j in range(D // LANES)]

    def compute_chunk(buf):
        def row_body(r, _):
            for j in range(D // LANES):
                sl = pl.ds(j * LANES, LANES)
                obuf[buf][r, sl] = rows[buf][r, sl] + tok[j]
            return 0

        lax.fori_loop(0, CHUNK, row_body, 0, unroll=2)

    g = [None] * NCHUNK
    s = [None] * NCHUNK
    g[0] = pltpu.async_copy(tab_hbm.at[idx_v.at[0]], rows[0], gsem[0])
    for c in range(NCHUNK):
        b = c % 2
        if c + 1 < NCHUNK:
            g[c + 1] = pltpu.async_copy(
                tab_hbm.at[idx_v.at[c + 1]], rows[(c + 1) % 2], gsem[(c + 1) % 2])
        g[c].wait()
        if c >= 2:
            s[c - 2].wait()
        compute_chunk(b)
        s[c] = pltpu.async_copy(
            obuf[b],
            out_hbm.at[pl.ds(row0 + c * CHUNK, CHUNK), pl.ds(0, D)],
            ssem[b])
    s[NCHUNK - 2].wait()
    s[NCHUNK - 1].wait()
    ecp.wait()


@jax.jit
def _pert_embed(idx2d, expr, tab, tok):
    mesh = plsc.VectorSubcoreMesh(core_axis_name="c", subcore_axis_name="s")
    run = functools.partial(
        pl.kernel, mesh=mesh,
        compiler_params=pltpu.CompilerParams(use_tc_tiling_on_sc=False),
        out_type=jax.ShapeDtypeStruct((BATCH, OD), jnp.float32),
        scratch_types=[
            pltpu.VMEM((NCHUNK, CHUNK), jnp.int32),
            pltpu.VMEM((D,), jnp.float32),
            pltpu.VMEM((CHUNK, D), jnp.float32),
            pltpu.VMEM((CHUNK, D), jnp.float32),
            pltpu.VMEM((CHUNK, D), jnp.float32),
            pltpu.VMEM((CHUNK, D), jnp.float32),
            pltpu.SemaphoreType.DMA,
            pltpu.SemaphoreType.DMA,
            pltpu.SemaphoreType.DMA,
            pltpu.SemaphoreType.DMA,
            pltpu.SemaphoreType.DMA,
        ],
    )(_sc_body)
    return run(idx2d, expr, tab, tok)


def kernel(pert_index, pert_expression, pos, pert_token):
    idx2d = pert_index.astype(jnp.int32).reshape(BATCH // CHUNK, CHUNK)
    tab = pos.reshape(VOCAB, D)
    out2d = _pert_embed(idx2d, pert_expression.reshape(BATCH, 1), tab, pert_token)
    return out2d.reshape(BATCH, 1, OD)
